# Initial kernel scaffold; baseline (speedup 1.0000x reference)
import jax
import jax.numpy as jnp
from jax.experimental import pallas as pl


def kernel(x, edge_index, batch, pre_W1, pre_b1, pre_W2, pre_b2, id_W, id_b, post_W1, post_b1, post_W2, post_b2, a00, a01, a10):
    raise NotImplementedError("write your pallas kernel here")


# speedup vs baseline: 3.0051x; 3.0051x over previous
import functools

import jax
import jax.numpy as jnp
from jax import lax
from jax.experimental import pallas as pl
from jax.experimental.pallas import tpu as pltpu
from jax.experimental.pallas import tpu_sc as plsc

N = 10000
E = 320000
D = 128
G = 128
C = 10

NC = 2
NS = 16
NW = NC * NS

CH = 128
CPT = 80
EPT = CH * CPT
EPAD = EPT * NW
NPAD = 10112
ZR = NPAD // NS

RB = 2000
NB = N // RB

_PREC = jax.lax.Precision.HIGHEST


def _mm(a, b):
    return jax.lax.dot_general(a, b, (((a.ndim - 1,), (0,)), ((), ())),
                               precision=_PREC, preferred_element_type=jnp.float32)


def _sc_agg_body(h_hbm, src_hbm, dst_hbm, z_hbm, out_hbm,
                 srcv0, dstv0, rows0, srcv1, dstv1, rows1, agg_sh,
                 sem0, sem1):
    cid = lax.axis_index("c")
    sid = lax.axis_index("s")
    wid = cid * NS + sid

    pltpu.sync_copy(z_hbm, agg_sh.at[pl.ds(sid * ZR, ZR)])
    plsc.subcore_barrier()

    base = wid * EPT

    pltpu.sync_copy(src_hbm.at[pl.ds(base, CH)], srcv0)
    pltpu.sync_copy(dst_hbm.at[pl.ds(base, CH)], dstv0)
    pltpu.async_copy(h_hbm.at[srcv0], rows0, sem0)

    def step(i, carry):
        off = base + 2 * i * CH
        pltpu.sync_copy(src_hbm.at[pl.ds(off + CH, CH)], srcv1)
        pltpu.sync_copy(dst_hbm.at[pl.ds(off + CH, CH)], dstv1)
        pltpu.async_copy(h_hbm.at[srcv1], rows1, sem1)
        pltpu.make_async_copy(h_hbm.at[srcv0], rows0, sem0).wait()
        pltpu.sync_copy(rows0, agg_sh.at[dstv0], add=True)

        @pl.when(i + 1 < CPT // 2)
        def _():
            pltpu.sync_copy(src_hbm.at[pl.ds(off + 2 * CH, CH)], srcv0)
            pltpu.sync_copy(dst_hbm.at[pl.ds(off + 2 * CH, CH)], dstv0)
            pltpu.async_copy(h_hbm.at[srcv0], rows0, sem0)

        pltpu.make_async_copy(h_hbm.at[srcv1], rows1, sem1).wait()
        pltpu.sync_copy(rows1, agg_sh.at[dstv1], add=True)
        return carry

    lax.fori_loop(0, CPT // 2, step, 0)
    plsc.subcore_barrier()

    pltpu.sync_copy(agg_sh.at[pl.ds(sid * ZR, ZR)],
                    out_hbm.at[pl.ds(cid * NPAD + sid * ZR, ZR)])


@functools.lru_cache(maxsize=1)
def _make_sc_agg():
    return pl.kernel(
        _sc_agg_body,
        mesh=plsc.VectorSubcoreMesh(core_axis_name="c", subcore_axis_name="s"),
        out_type=jax.ShapeDtypeStruct((2 * NPAD, D), jnp.float32),
        scratch_types=[
        pltpu.VMEM((CH,), jnp.int32),
        pltpu.VMEM((CH,), jnp.int32),
        pltpu.VMEM((CH, D), jnp.float32),
        pltpu.VMEM((CH,), jnp.int32),
        pltpu.VMEM((CH,), jnp.int32),
        pltpu.VMEM((CH, D), jnp.float32),
            pltpu.VMEM_SHARED((NPAD, D), jnp.float32),
            pltpu.SemaphoreType.DMA,
            pltpu.SemaphoreType.DMA,
        ],
    )


def _sc_agg(h, src_p, dst_p, zrows):
    return _make_sc_agg()(h, src_p, dst_p, zrows).reshape(2, NPAD, D)


def _premlp_body(x_ref, w1_ref, b1_ref, w2_ref, b2_ref, o_ref):
    h = jnp.maximum(_mm(x_ref[...], w1_ref[...]) + b1_ref[...], 0.0)
    o_ref[...] = _mm(h, w2_ref[...]) + b2_ref[...]


def _stage1_body(h_ref, p0_ref, p1_ref, w_ref, b_ref, o_ref):
    s = h_ref[...] + p0_ref[0] + p1_ref[0]
    o_ref[...] = jnp.maximum(_mm(s, w_ref[...]) + b_ref[...], 0.0)


def _stage2_body(h_ref, p0_ref, p1_ref, w_ref, b_ref, c_ref, i1_ref, h2_ref):
    s = h_ref[...] + p0_ref[0] + p1_ref[0]
    i1 = jnp.maximum(_mm(s, w_ref[...]) + b_ref[...], 0.0)
    i1_ref[...] = i1
    h2_ref[...] = c_ref[0, 0] * h_ref[...] + i1


def _stage3_body(h_ref, p0_ref, p1_ref, w_ref, b_ref, i0_ref, i1_ref,
                 ca_ref, cb_ref, o_ref):
    s = h_ref[...] + p0_ref[0] + p1_ref[0]
    s2l = jnp.maximum(_mm(s, w_ref[...]) + b_ref[...], 0.0)
    o_ref[...] = ca_ref[0, 0] * i0_ref[...] + cb_ref[0, 0] * i1_ref[...] + s2l


def _final_body(h_ref, p0_ref, p1_ref, w_ref, b_ref, batch_ref,
                pw1_ref, pb1_ref, pw2_ref, pb2_ref, y_ref, emb_ref):
    i = pl.program_id(0)
    s = h_ref[...] + p0_ref[0] + p1_ref[0]
    last = jnp.maximum(_mm(s, w_ref[...]) + b_ref[...], 0.0)
    gids = lax.broadcasted_iota(jnp.int32, (G, RB), 0)
    onehot = jnp.where(gids == batch_ref[0], 1.0, 0.0)
    part = _mm(onehot, last)

    @pl.when(i == 0)
    def _():
        emb_ref[...] = part

    @pl.when(i > 0)
    def _():
        emb_ref[...] = emb_ref[...] + part

    @pl.when(i == NB - 1)
    def _():
        hh = jnp.maximum(_mm(emb_ref[...], pw1_ref[...]) + pb1_ref[...], 0.0)
        y_ref[...] = _mm(hh, pw2_ref[...]) + pb2_ref[...]


def _row_spec():
    return pl.BlockSpec((RB, D), lambda i: (i, 0))


def _p0_spec():
    return pl.BlockSpec((1, RB, D), lambda i: (0, i, 0))


def _p1_spec():
    return pl.BlockSpec((1, RB, D), lambda i: (1, i, 0))


def _full_spec(shape):
    nd = len(shape)
    return pl.BlockSpec(shape, lambda i: (0,) * nd)


def _premlp(x, w1, b1, w2, b2):
    return pl.pallas_call(
        _premlp_body,
        grid=(NB,),
        in_specs=[_row_spec(), _full_spec((D, D)), _full_spec((1, D)),
                  _full_spec((D, D)), _full_spec((1, D))],
        out_specs=_row_spec(),
        out_shape=jax.ShapeDtypeStruct((N, D), jnp.float32),
    )(x, w1, b1, w2, b2)


def _stage1(h, agg, w, b):
    return pl.pallas_call(
        _stage1_body,
        grid=(NB,),
        in_specs=[_row_spec(), _p0_spec(), _p1_spec(),
                  _full_spec((D, D)), _full_spec((1, D))],
        out_specs=_row_spec(),
        out_shape=jax.ShapeDtypeStruct((N, D), jnp.float32),
    )(h, agg, agg, w, b)


def _stage2(h, agg, w, b, c00):
    return pl.pallas_call(
        _stage2_body,
        grid=(NB,),
        in_specs=[_row_spec(), _p0_spec(), _p1_spec(),
                  _full_spec((D, D)), _full_spec((1, D)), _full_spec((1, 1))],
        out_specs=[_row_spec(), _row_spec()],
        out_shape=[jax.ShapeDtypeStruct((N, D), jnp.float32),
                   jax.ShapeDtypeStruct((N, D), jnp.float32)],
    )(h, agg, agg, w, b, c00)


def _stage3(h, agg, w, b, i0, i1, c01, c10):
    return pl.pallas_call(
        _stage3_body,
        grid=(NB,),
        in_specs=[_row_spec(), _p0_spec(), _p1_spec(),
                  _full_spec((D, D)), _full_spec((1, D)),
                  _row_spec(), _row_spec(), _full_spec((1, 1)), _full_spec((1, 1))],
        out_specs=_row_spec(),
        out_shape=jax.ShapeDtypeStruct((N, D), jnp.float32),
    )(h, agg, agg, w, b, i0, i1, c01, c10)


def _final(h, agg, w, b, batch3, pw1, pb1, pw2, pb2):
    return pl.pallas_call(
        _final_body,
        grid=(NB,),
        in_specs=[_row_spec(), _p0_spec(), _p1_spec(),
                  _full_spec((D, D)), _full_spec((1, D)),
                  pl.BlockSpec((1, 1, RB), lambda i: (i, 0, 0)),
                  _full_spec((D, D)), _full_spec((1, D)),
                  _full_spec((D, C)), _full_spec((1, C))],
        out_specs=_full_spec((G, C)),
        out_shape=jax.ShapeDtypeStruct((G, C), jnp.float32),
        scratch_shapes=[pltpu.VMEM((G, D), jnp.float32)],
    )(h, agg, agg, w, b, batch3, pw1, pb1, pw2, pb2)


def _soft1(a):
    m = jnp.maximum(a[0], a[1])
    e0 = jnp.exp(a[0] - m)
    e1 = jnp.exp(a[1] - m)
    return (e1 / (e0 + e1)).reshape(1, 1)


def kernel(x, edge_index, batch, pre_W1, pre_b1, pre_W2, pre_b2, id_W, id_b,
           post_W1, post_b1, post_W2, post_b2, a00, a01, a10):
    src = edge_index[0]
    dst = edge_index[1]
    pad = EPAD - E
    src_p = jnp.concatenate([src, jnp.zeros((pad,), jnp.int32)])
    dst_p = jnp.concatenate([dst, jnp.full((pad,), N, jnp.int32)])
    zrows = jnp.zeros((ZR, D), jnp.float32)
    batch3 = batch.reshape(NB, 1, RB)

    b_pre1 = pre_b1.reshape(1, D)
    b_pre2 = pre_b2.reshape(1, D)
    b_id = id_b.reshape(1, D)
    b_post1 = post_b1.reshape(1, D)
    b_post2 = post_b2.reshape(1, C)

    c00 = _soft1(a00)
    c01 = _soft1(a01)
    c10 = _soft1(a10)

    in0 = _premlp(x, pre_W1, b_pre1, pre_W2, b_pre2)
    agg0 = _sc_agg(in0, src_p, dst_p, zrows)
    i0 = _stage1(in0, agg0, id_W, b_id)
    agg1 = _sc_agg(i0, src_p, dst_p, zrows)
    i1, h2 = _stage2(i0, agg1, id_W, b_id, c00)
    agg2 = _sc_agg(h2, src_p, dst_p, zrows)
    h3 = _stage3(h2, agg2, id_W, b_id, i0, i1, c01, c10)
    agg3 = _sc_agg(h3, src_p, dst_p, zrows)
    y = _final(h3, agg3, id_W, b_id, batch3, post_W1, b_post1, post_W2, b_post2)
    return y


# speedup vs baseline: 3.0086x; 1.0012x over previous
import functools

import jax
import jax.numpy as jnp
from jax import lax
from jax.experimental import pallas as pl
from jax.experimental.pallas import tpu as pltpu
from jax.experimental.pallas import tpu_sc as plsc

N = 10000
E = 320000
D = 128
G = 128
C = 10

NC = 2
NS = 16
NW = NC * NS

CH = 128
CPT = 80
NBUF = 2
HCPT = 40
EPT = CH * CPT
EPAD = EPT * NW
NPAD = 10112
ZR = NPAD // NS

RB = 2000
NB = N // RB

_PREC = jax.lax.Precision.HIGHEST


def _mm(a, b):
    return jax.lax.dot_general(a, b, (((a.ndim - 1,), (0,)), ((), ())),
                               precision=_PREC, preferred_element_type=jnp.float32)


def _sc_agg_body(h_hbm, src_hbm, dst_hbm, z_hbm, out_hbm,
                 idx_s, idx_d, rows, agg_sh, sem0, sem1):
    cid = lax.axis_index("c")
    sid = lax.axis_index("s")
    wid = cid * NS + sid
    sems = (sem0, sem1)

    pltpu.sync_copy(z_hbm, agg_sh.at[pl.ds(sid * ZR, ZR)])
    plsc.subcore_barrier()

    for h0 in (0, HCPT):
        pltpu.sync_copy(src_hbm.at[wid, pl.ds(h0, HCPT)], idx_s)
        pltpu.sync_copy(dst_hbm.at[wid, pl.ds(h0, HCPT)], idx_d)

        for b in range(NBUF):
            pltpu.async_copy(h_hbm.at[idx_s.at[b]], rows.at[b], sems[b])

        def step(i, carry):
            j0 = i * NBUF
            for b in range(NBUF):
                j = j0 + b
                pltpu.make_async_copy(h_hbm.at[idx_s.at[j]], rows.at[b],
                                      sems[b]).wait()
                pltpu.sync_copy(rows.at[b], agg_sh.at[idx_d.at[j]], add=True)

                @pl.when(j + NBUF < HCPT)
                def _():
                    pltpu.async_copy(h_hbm.at[idx_s.at[j + NBUF]], rows.at[b],
                                     sems[b])
            return carry

        lax.fori_loop(0, HCPT // NBUF, step, 0)

    plsc.subcore_barrier()

    pltpu.sync_copy(agg_sh.at[pl.ds(sid * ZR, ZR)],
                    out_hbm.at[pl.ds(cid * NPAD + sid * ZR, ZR)])


@functools.lru_cache(maxsize=1)
def _make_sc_agg():
    return pl.kernel(
        _sc_agg_body,
        mesh=plsc.VectorSubcoreMesh(core_axis_name="c", subcore_axis_name="s"),
        out_type=jax.ShapeDtypeStruct((2 * NPAD, D), jnp.float32),
        scratch_types=[
            pltpu.VMEM((HCPT, CH), jnp.int32),
            pltpu.VMEM((HCPT, CH), jnp.int32),
            pltpu.VMEM((NBUF, CH, D), jnp.float32),
            pltpu.VMEM_SHARED((NPAD, D), jnp.float32),
            pltpu.SemaphoreType.DMA,
            pltpu.SemaphoreType.DMA,
        ],
    )


def _sc_agg(h, src_p, dst_p, zrows):
    return _make_sc_agg()(h, src_p, dst_p, zrows).reshape(2, NPAD, D)


def _premlp_body(x_ref, w1_ref, b1_ref, w2_ref, b2_ref, o_ref):
    h = jnp.maximum(_mm(x_ref[...], w1_ref[...]) + b1_ref[...], 0.0)
    o_ref[...] = _mm(h, w2_ref[...]) + b2_ref[...]


def _stage1_body(h_ref, p0_ref, p1_ref, w_ref, b_ref, o_ref):
    s = h_ref[...] + p0_ref[0] + p1_ref[0]
    o_ref[...] = jnp.maximum(_mm(s, w_ref[...]) + b_ref[...], 0.0)


def _stage2_body(h_ref, p0_ref, p1_ref, w_ref, b_ref, c_ref, i1_ref, h2_ref):
    s = h_ref[...] + p0_ref[0] + p1_ref[0]
    i1 = jnp.maximum(_mm(s, w_ref[...]) + b_ref[...], 0.0)
    i1_ref[...] = i1
    h2_ref[...] = c_ref[0, 0] * h_ref[...] + i1


def _stage3_body(h_ref, p0_ref, p1_ref, w_ref, b_ref, i0_ref, i1_ref,
                 ca_ref, cb_ref, o_ref):
    s = h_ref[...] + p0_ref[0] + p1_ref[0]
    s2l = jnp.maximum(_mm(s, w_ref[...]) + b_ref[...], 0.0)
    o_ref[...] = ca_ref[0, 0] * i0_ref[...] + cb_ref[0, 0] * i1_ref[...] + s2l


def _final_body(h_ref, p0_ref, p1_ref, w_ref, b_ref, batch_ref,
                pw1_ref, pb1_ref, pw2_ref, pb2_ref, y_ref, emb_ref):
    i = pl.program_id(0)
    s = h_ref[...] + p0_ref[0] + p1_ref[0]
    last = jnp.maximum(_mm(s, w_ref[...]) + b_ref[...], 0.0)
    gids = lax.broadcasted_iota(jnp.int32, (G, RB), 0)
    onehot = jnp.where(gids == batch_ref[0], 1.0, 0.0)
    part = _mm(onehot, last)

    @pl.when(i == 0)
    def _():
        emb_ref[...] = part

    @pl.when(i > 0)
    def _():
        emb_ref[...] = emb_ref[...] + part

    @pl.when(i == NB - 1)
    def _():
        hh = jnp.maximum(_mm(emb_ref[...], pw1_ref[...]) + pb1_ref[...], 0.0)
        y_ref[...] = _mm(hh, pw2_ref[...]) + pb2_ref[...]


def _row_spec():
    return pl.BlockSpec((RB, D), lambda i: (i, 0))


def _p0_spec():
    return pl.BlockSpec((1, RB, D), lambda i: (0, i, 0))


def _p1_spec():
    return pl.BlockSpec((1, RB, D), lambda i: (1, i, 0))


def _full_spec(shape):
    nd = len(shape)
    return pl.BlockSpec(shape, lambda i: (0,) * nd)


def _premlp(x, w1, b1, w2, b2):
    return pl.pallas_call(
        _premlp_body,
        grid=(NB,),
        in_specs=[_row_spec(), _full_spec((D, D)), _full_spec((1, D)),
                  _full_spec((D, D)), _full_spec((1, D))],
        out_specs=_row_spec(),
        out_shape=jax.ShapeDtypeStruct((N, D), jnp.float32),
    )(x, w1, b1, w2, b2)


def _stage1(h, agg, w, b):
    return pl.pallas_call(
        _stage1_body,
        grid=(NB,),
        in_specs=[_row_spec(), _p0_spec(), _p1_spec(),
                  _full_spec((D, D)), _full_spec((1, D))],
        out_specs=_row_spec(),
        out_shape=jax.ShapeDtypeStruct((N, D), jnp.float32),
    )(h, agg, agg, w, b)


def _stage2(h, agg, w, b, c00):
    return pl.pallas_call(
        _stage2_body,
        grid=(NB,),
        in_specs=[_row_spec(), _p0_spec(), _p1_spec(),
                  _full_spec((D, D)), _full_spec((1, D)), _full_spec((1, 1))],
        out_specs=[_row_spec(), _row_spec()],
        out_shape=[jax.ShapeDtypeStruct((N, D), jnp.float32),
                   jax.ShapeDtypeStruct((N, D), jnp.float32)],
    )(h, agg, agg, w, b, c00)


def _stage3(h, agg, w, b, i0, i1, c01, c10):
    return pl.pallas_call(
        _stage3_body,
        grid=(NB,),
        in_specs=[_row_spec(), _p0_spec(), _p1_spec(),
                  _full_spec((D, D)), _full_spec((1, D)),
                  _row_spec(), _row_spec(), _full_spec((1, 1)), _full_spec((1, 1))],
        out_specs=_row_spec(),
        out_shape=jax.ShapeDtypeStruct((N, D), jnp.float32),
    )(h, agg, agg, w, b, i0, i1, c01, c10)


def _final(h, agg, w, b, batch3, pw1, pb1, pw2, pb2):
    return pl.pallas_call(
        _final_body,
        grid=(NB,),
        in_specs=[_row_spec(), _p0_spec(), _p1_spec(),
                  _full_spec((D, D)), _full_spec((1, D)),
                  pl.BlockSpec((1, 1, RB), lambda i: (i, 0, 0)),
                  _full_spec((D, D)), _full_spec((1, D)),
                  _full_spec((D, C)), _full_spec((1, C))],
        out_specs=_full_spec((G, C)),
        out_shape=jax.ShapeDtypeStruct((G, C), jnp.float32),
        scratch_shapes=[pltpu.VMEM((G, D), jnp.float32)],
    )(h, agg, agg, w, b, batch3, pw1, pb1, pw2, pb2)


def _soft1(a):
    m = jnp.maximum(a[0], a[1])
    e0 = jnp.exp(a[0] - m)
    e1 = jnp.exp(a[1] - m)
    return (e1 / (e0 + e1)).reshape(1, 1)


def kernel(x, edge_index, batch, pre_W1, pre_b1, pre_W2, pre_b2, id_W, id_b,
           post_W1, post_b1, post_W2, post_b2, a00, a01, a10):
    src = edge_index[0]
    dst = edge_index[1]
    pad = EPAD - E
    src_p = jnp.concatenate([src, jnp.zeros((pad,), jnp.int32)]).reshape(NW, CPT, CH)
    dst_p = jnp.concatenate([dst, jnp.full((pad,), N, jnp.int32)]).reshape(NW, CPT, CH)
    zrows = jnp.zeros((ZR, D), jnp.float32)
    batch3 = batch.reshape(NB, 1, RB)

    b_pre1 = pre_b1.reshape(1, D)
    b_pre2 = pre_b2.reshape(1, D)
    b_id = id_b.reshape(1, D)
    b_post1 = post_b1.reshape(1, D)
    b_post2 = post_b2.reshape(1, C)

    c00 = _soft1(a00)
    c01 = _soft1(a01)
    c10 = _soft1(a10)

    in0 = _premlp(x, pre_W1, b_pre1, pre_W2, b_pre2)
    agg0 = _sc_agg(in0, src_p, dst_p, zrows)
    i0 = _stage1(in0, agg0, id_W, b_id)
    agg1 = _sc_agg(i0, src_p, dst_p, zrows)
    i1, h2 = _stage2(i0, agg1, id_W, b_id, c00)
    agg2 = _sc_agg(h2, src_p, dst_p, zrows)
    h3 = _stage3(h2, agg2, id_W, b_id, i0, i1, c01, c10)
    agg3 = _sc_agg(h3, src_p, dst_p, zrows)
    y = _final(h3, agg3, id_W, b_id, batch3, post_W1, b_post1, post_W2, b_post2)
    return y
